# baseline (speedup 1.0000x reference)
import functools

import jax
import jax.numpy as jnp
from jax import lax
from jax.experimental import pallas as pl
from jax.experimental.pallas import tpu as pltpu
from jax.experimental.pallas import tpu_sc as plsc

_L = 16
_NC = 2
_NW = 32


def _tc_body(hs_ref, tg_ref, out_hs_ref, stats_ref):
    h = hs_ref[...]
    t = tg_ref[...]
    out_hs_ref[...] = h
    d = h - t
    stats_ref[0, :] = jnp.sum(d * d, axis=1)
    stats_ref[1, :] = jnp.sum(jnp.abs(h), axis=1)
    stats_ref[2, :] = jnp.sum(t, axis=1)


def _tc_stage(hs, tg, rows_per_block=1024):
    n, d = hs.shape
    grid = (n // rows_per_block,)
    return pl.pallas_call(
        _tc_body,
        grid=grid,
        in_specs=[
            pl.BlockSpec((rows_per_block, d), lambda i: (i, 0)),
            pl.BlockSpec((rows_per_block, d), lambda i: (i, 0)),
        ],
        out_specs=[
            pl.BlockSpec((rows_per_block, d), lambda i: (i, 0)),
            pl.BlockSpec((3, rows_per_block), lambda i: (0, i)),
        ],
        out_shape=[
            jax.ShapeDtypeStruct((n, d), jnp.float32),
            jax.ShapeDtypeStruct((3, n), jnp.float32),
        ],
    )(hs, tg)


@functools.cache
def _make_sc_compact(n):
    rows_per_sub = n // _NW
    iters = rows_per_sub // _L
    mesh = plsc.VectorSubcoreMesh(
        core_axis_name="c", subcore_axis_name="s")

    @functools.partial(
        pl.kernel,
        out_type=jax.ShapeDtypeStruct((_NW, 4 * _L), jnp.float32),
        mesh=mesh,
        scratch_types=[
            pltpu.VMEM((rows_per_sub,), jnp.float32),
            pltpu.VMEM((rows_per_sub,), jnp.float32),
            pltpu.VMEM((rows_per_sub,), jnp.float32),
            pltpu.VMEM((4 * _L,), jnp.float32),
        ],
    )
    def sc_compact(stats_hbm, out_hbm, sq_v, ab_v, ts_v, part_v):
        zero = jnp.zeros((_L,), jnp.float32)
        one = jnp.ones((_L,), jnp.float32)
        wid = lax.axis_index("s") * _NC + lax.axis_index("c")
        base = wid * rows_per_sub
        pltpu.sync_copy(stats_hbm.at[pl.ds(0 * n + base, rows_per_sub)], sq_v)
        pltpu.sync_copy(stats_hbm.at[pl.ds(1 * n + base, rows_per_sub)], ab_v)
        pltpu.sync_copy(stats_hbm.at[pl.ds(2 * n + base, rows_per_sub)], ts_v)

        def body(i, carry):
            acc_sq, acc_ab, acc_ct = carry
            m = ts_v[pl.ds(i * _L, _L)] != 0.0
            acc_sq = acc_sq + jnp.where(m, sq_v[pl.ds(i * _L, _L)], zero)
            acc_ab = acc_ab + jnp.where(m, ab_v[pl.ds(i * _L, _L)], zero)
            acc_ct = acc_ct + jnp.where(m, one, zero)
            return (acc_sq, acc_ab, acc_ct)

        acc_sq, acc_ab, acc_ct = lax.fori_loop(
            0, iters, body, (zero, zero, zero))
        part_v[pl.ds(0, _L)] = acc_sq
        part_v[pl.ds(_L, _L)] = acc_ab
        part_v[pl.ds(2 * _L, _L)] = acc_ct
        part_v[pl.ds(3 * _L, _L)] = zero
        pltpu.sync_copy(part_v, out_hbm.at[wid])

    return sc_compact


def kernel(hidden_states, targets):
    B, S, D = hidden_states.shape
    n = B * S
    hs = hidden_states.reshape(n, D)
    tg = targets.reshape(n, D)
    out_hs, stats = _tc_stage(hs, tg)
    parts = _make_sc_compact(n)(stats.reshape(3 * n))
    sq_tot = jnp.sum(parts[:, 0 * _L:1 * _L])
    ab_tot = jnp.sum(parts[:, 1 * _L:2 * _L])
    n_elems = jnp.sum(parts[:, 2 * _L:3 * _L]) * D
    return (sq_tot / n_elems, ab_tot / n_elems, out_hs.reshape(B, S, D))


# speedup vs baseline: 1.0247x; 1.0247x over previous
import functools

import jax
import jax.numpy as jnp
from jax import lax
from jax.experimental import pallas as pl
from jax.experimental.pallas import tpu as pltpu
from jax.experimental.pallas import tpu_sc as plsc

_L = 16
_NSUB = 16


def _tc_body(hs_ref, tg_ref, out_hs_ref, stats_ref):
    h = hs_ref[...]
    t = tg_ref[...]
    out_hs_ref[...] = h
    d = h - t
    stats_ref[0, :] = jnp.sum(d * d, axis=1)
    stats_ref[1, :] = jnp.sum(jnp.abs(h), axis=1)
    stats_ref[2, :] = jnp.sum(t, axis=1)


def _tc_stage(hs, tg, rows_per_block=1024):
    n, d = hs.shape
    grid = (n // rows_per_block,)
    return pl.pallas_call(
        _tc_body,
        grid=grid,
        in_specs=[
            pl.BlockSpec((rows_per_block, d), lambda i: (i, 0)),
            pl.BlockSpec((rows_per_block, d), lambda i: (i, 0)),
        ],
        out_specs=[
            pl.BlockSpec((rows_per_block, d), lambda i: (i, 0)),
            pl.BlockSpec((3, rows_per_block), lambda i: (0, i)),
        ],
        out_shape=[
            jax.ShapeDtypeStruct((n, d), jnp.float32),
            jax.ShapeDtypeStruct((3, n), jnp.float32),
        ],
    )(hs, tg)


@functools.cache
def _make_sc_compact(n):
    rows_per_sub = n // _NSUB
    iters = rows_per_sub // _L
    mesh = plsc.VectorSubcoreMesh(
        core_axis_name="c", subcore_axis_name="s", num_cores=1)

    @functools.partial(
        pl.kernel,
        out_type=jax.ShapeDtypeStruct((_NSUB, 4 * _L), jnp.float32),
        mesh=mesh,
        scratch_types=[
            pltpu.VMEM((rows_per_sub,), jnp.float32),
            pltpu.VMEM((rows_per_sub,), jnp.float32),
            pltpu.VMEM((rows_per_sub,), jnp.float32),
            pltpu.VMEM((4 * _L,), jnp.float32),
        ],
    )
    def sc_compact(stats_hbm, out_hbm, sq_v, ab_v, ts_v, part_v):
        zero = jnp.zeros((_L,), jnp.float32)
        one = jnp.ones((_L,), jnp.float32)
        sid = lax.axis_index("s")
        base = sid * rows_per_sub
        pltpu.sync_copy(stats_hbm.at[pl.ds(0 * n + base, rows_per_sub)], sq_v)
        pltpu.sync_copy(stats_hbm.at[pl.ds(1 * n + base, rows_per_sub)], ab_v)
        pltpu.sync_copy(stats_hbm.at[pl.ds(2 * n + base, rows_per_sub)], ts_v)

        def body(i, carry):
            acc_sq, acc_ab, acc_ct = carry
            m = ts_v[pl.ds(i * _L, _L)] != 0.0
            acc_sq = acc_sq + jnp.where(m, sq_v[pl.ds(i * _L, _L)], zero)
            acc_ab = acc_ab + jnp.where(m, ab_v[pl.ds(i * _L, _L)], zero)
            acc_ct = acc_ct + jnp.where(m, one, zero)
            return (acc_sq, acc_ab, acc_ct)

        acc_sq, acc_ab, acc_ct = lax.fori_loop(
            0, iters, body, (zero, zero, zero))
        part_v[pl.ds(0, _L)] = acc_sq
        part_v[pl.ds(_L, _L)] = acc_ab
        part_v[pl.ds(2 * _L, _L)] = acc_ct
        part_v[pl.ds(3 * _L, _L)] = zero
        pltpu.sync_copy(part_v, out_hbm.at[sid])

    return sc_compact


def kernel(hidden_states, targets):
    B, S, D = hidden_states.shape
    n = B * S
    hs = hidden_states.reshape(n, D)
    tg = targets.reshape(n, D)
    out_hs, stats = _tc_stage(hs, tg)
    parts = _make_sc_compact(n)(stats.reshape(3 * n))
    sq_tot = jnp.sum(parts[:, 0 * _L:1 * _L])
    ab_tot = jnp.sum(parts[:, 1 * _L:2 * _L])
    n_elems = jnp.sum(parts[:, 2 * _L:3 * _L]) * D
    return (sq_tot / n_elems, ab_tot / n_elems, out_hs.reshape(B, S, D))


# speedup vs baseline: 1.0294x; 1.0046x over previous
import functools

import jax
import jax.numpy as jnp
from jax import lax
from jax.experimental import pallas as pl
from jax.experimental.pallas import tpu as pltpu
from jax.experimental.pallas import tpu_sc as plsc

_L = 16
_NSUB = 16


def _tc_body(hs_ref, tg_ref, out_hs_ref, stats_ref):
    h = hs_ref[...]
    t = tg_ref[...]
    out_hs_ref[...] = h
    d = h - t
    stats_ref[0, 0, :] = jnp.sum(d * d, axis=1)
    stats_ref[0, 1, :] = jnp.sum(jnp.abs(h), axis=1)
    stats_ref[0, 2, :] = jnp.sum(t, axis=1)


def _tc_stage(hs, tg, rows_per_block=1024):
    n, d = hs.shape
    grid = (n // rows_per_block,)
    return pl.pallas_call(
        _tc_body,
        grid=grid,
        in_specs=[
            pl.BlockSpec((rows_per_block, d), lambda i: (i, 0)),
            pl.BlockSpec((rows_per_block, d), lambda i: (i, 0)),
        ],
        out_specs=[
            pl.BlockSpec((rows_per_block, d), lambda i: (i, 0)),
            pl.BlockSpec((1, 3, rows_per_block), lambda i: (i, 0, 0)),
        ],
        out_shape=[
            jax.ShapeDtypeStruct((n, d), jnp.float32),
            jax.ShapeDtypeStruct((n // rows_per_block, 3, rows_per_block),
                                 jnp.float32),
        ],
    )(hs, tg)


@functools.cache
def _make_sc_compact(n):
    rows_per_sub = n // _NSUB
    iters = rows_per_sub // _L
    mesh = plsc.VectorSubcoreMesh(
        core_axis_name="c", subcore_axis_name="s", num_cores=1)

    @functools.partial(
        pl.kernel,
        out_type=jax.ShapeDtypeStruct((_NSUB, 4 * _L), jnp.float32),
        mesh=mesh,
        scratch_types=[
            pltpu.VMEM((3 * rows_per_sub,), jnp.float32),
            pltpu.VMEM((4 * _L,), jnp.float32),
        ],
    )
    def sc_compact(stats_hbm, out_hbm, sl_v, part_v):
        zero = jnp.zeros((_L,), jnp.float32)
        one = jnp.ones((_L,), jnp.float32)
        sid = lax.axis_index("s")
        base = sid * 3 * rows_per_sub
        pltpu.sync_copy(stats_hbm.at[pl.ds(base, 3 * rows_per_sub)], sl_v)

        def body(i, carry):
            acc_sq, acc_ab, acc_ct = carry
            m = sl_v[pl.ds(2 * rows_per_sub + i * _L, _L)] != 0.0
            acc_sq = acc_sq + jnp.where(m, sl_v[pl.ds(i * _L, _L)], zero)
            acc_ab = acc_ab + jnp.where(
                m, sl_v[pl.ds(rows_per_sub + i * _L, _L)], zero)
            acc_ct = acc_ct + jnp.where(m, one, zero)
            return (acc_sq, acc_ab, acc_ct)

        acc_sq, acc_ab, acc_ct = lax.fori_loop(
            0, iters, body, (zero, zero, zero))
        part_v[pl.ds(0, _L)] = acc_sq
        part_v[pl.ds(_L, _L)] = acc_ab
        part_v[pl.ds(2 * _L, _L)] = acc_ct
        part_v[pl.ds(3 * _L, _L)] = zero
        pltpu.sync_copy(part_v, out_hbm.at[sid])

    return sc_compact


def kernel(hidden_states, targets):
    B, S, D = hidden_states.shape
    n = B * S
    hs = hidden_states.reshape(n, D)
    tg = targets.reshape(n, D)
    out_hs, stats = _tc_stage(hs, tg)
    parts = _make_sc_compact(n)(stats.reshape(3 * n))
    sq_tot = jnp.sum(parts[:, 0 * _L:1 * _L])
    ab_tot = jnp.sum(parts[:, 1 * _L:2 * _L])
    n_elems = jnp.sum(parts[:, 2 * _L:3 * _L]) * D
    return (sq_tot / n_elems, ab_tot / n_elems, out_hs.reshape(B, S, D))


# speedup vs baseline: 1.0345x; 1.0050x over previous
import functools

import jax
import jax.numpy as jnp
from jax import lax
from jax.experimental import pallas as pl
from jax.experimental.pallas import tpu as pltpu
from jax.experimental.pallas import tpu_sc as plsc

_L = 16
_NSUB = 16


def _tc_body(hs_ref, tg_ref, out_hs_ref, stats_ref):
    h = hs_ref[...]
    t = tg_ref[...]
    out_hs_ref[...] = h
    d = h - t
    stats_ref[0, 0, :] = jnp.sum(d * d, axis=1)
    stats_ref[0, 1, :] = jnp.sum(jnp.abs(h), axis=1)
    stats_ref[0, 2, :] = jnp.sum(t, axis=1)


def _tc_stage(hs, tg, rows_per_block=1024):
    n, d = hs.shape
    grid = (n // rows_per_block,)
    return pl.pallas_call(
        _tc_body,
        grid=grid,
        in_specs=[
            pl.BlockSpec((rows_per_block, d), lambda i: (i, 0)),
            pl.BlockSpec((rows_per_block, d), lambda i: (i, 0)),
        ],
        out_specs=[
            pl.BlockSpec((rows_per_block, d), lambda i: (i, 0)),
            pl.BlockSpec((1, 3, rows_per_block), lambda i: (i, 0, 0)),
        ],
        out_shape=[
            jax.ShapeDtypeStruct((n, d), jnp.float32),
            jax.ShapeDtypeStruct((n // rows_per_block, 3, rows_per_block),
                                 jnp.float32),
        ],
    )(hs, tg)


@functools.cache
def _make_sc_compact(n):
    rows_per_sub = n // _NSUB
    iters = rows_per_sub // _L
    mesh = plsc.VectorSubcoreMesh(
        core_axis_name="c", subcore_axis_name="s", num_cores=1)

    @functools.partial(
        pl.kernel,
        out_type=jax.ShapeDtypeStruct((_NSUB, 4 * _L), jnp.float32),
        mesh=mesh,
        scratch_types=[
            pltpu.VMEM((3 * rows_per_sub,), jnp.float32),
            pltpu.VMEM((4 * _L,), jnp.float32),
        ],
    )
    def sc_compact(stats_hbm, out_hbm, sl_v, part_v):
        zero = jnp.zeros((_L,), jnp.float32)
        one = jnp.ones((_L,), jnp.float32)
        sid = lax.axis_index("s")
        base = sid * 3 * rows_per_sub
        pltpu.sync_copy(stats_hbm.at[pl.ds(base, 3 * rows_per_sub)], sl_v)

        def body(i, carry):
            acc_sq, acc_ab, acc_ct = carry
            m = sl_v[pl.ds(2 * rows_per_sub + i * _L, _L)] != 0.0
            acc_sq = acc_sq + jnp.where(m, sl_v[pl.ds(i * _L, _L)], zero)
            acc_ab = acc_ab + jnp.where(
                m, sl_v[pl.ds(rows_per_sub + i * _L, _L)], zero)
            acc_ct = acc_ct + jnp.where(m, one, zero)
            return (acc_sq, acc_ab, acc_ct)

        acc_sq, acc_ab, acc_ct = lax.fori_loop(
            0, iters, body, (zero, zero, zero), unroll=4)
        part_v[pl.ds(0, _L)] = acc_sq
        part_v[pl.ds(_L, _L)] = acc_ab
        part_v[pl.ds(2 * _L, _L)] = acc_ct
        part_v[pl.ds(3 * _L, _L)] = zero
        pltpu.sync_copy(part_v, out_hbm.at[sid])

    return sc_compact


def kernel(hidden_states, targets):
    B, S, D = hidden_states.shape
    n = B * S
    hs = hidden_states.reshape(n, D)
    tg = targets.reshape(n, D)
    out_hs, stats = _tc_stage(hs, tg)
    parts = _make_sc_compact(n)(stats.reshape(3 * n))
    sq_tot = jnp.sum(parts[:, 0 * _L:1 * _L])
    ab_tot = jnp.sum(parts[:, 1 * _L:2 * _L])
    n_elems = jnp.sum(parts[:, 2 * _L:3 * _L]) * D
    return (sq_tot / n_elems, ab_tot / n_elems, out_hs.reshape(B, S, D))
